# baseline (speedup 1.0000x reference)
import functools

import jax
import jax.numpy as jnp
from jax import lax
from jax.experimental import pallas as pl
from jax.experimental.pallas import tpu as pltpu
from jax.experimental.pallas import tpu_sc as plsc

NC = 2
NS = 16
NW = NC * NS

D = 64
CHUNK = 128


def _make_gather(n_rows: int):
  assert n_rows % (NW * CHUNK) == 0
  per_w = n_rows // NW
  n_chunks = per_w // CHUNK

  mesh = plsc.VectorSubcoreMesh(
      core_axis_name="c", subcore_axis_name="s", num_cores=NC,
      num_subcores=NS)

  @functools.partial(
      pl.kernel,
      out_type=jax.ShapeDtypeStruct((n_rows, D), jnp.float32),
      mesh=mesh,
      compiler_params=pltpu.CompilerParams(use_tc_tiling_on_sc=False),
      scratch_types=[
          pltpu.VMEM((per_w,), jnp.int32),
          pltpu.VMEM((CHUNK, D), jnp.float32),
          pltpu.VMEM((CHUNK, D), jnp.float32),
          pltpu.SemaphoreType.DMA,
          pltpu.SemaphoreType.DMA,
      ],
  )
  def gather_kernel(idx_hbm, table_hbm, out_hbm, idx_v, rows0, rows1,
                    sem0, sem1):
    wid = lax.axis_index("s") * NC + lax.axis_index("c")
    base = wid * per_w
    pltpu.sync_copy(idx_hbm.at[pl.ds(base, per_w)], idx_v)

    rows = (rows0, rows1)
    sems = (sem0, sem1)

    def start(g, buf):
      pltpu.async_copy(
          table_hbm.at[idx_v.at[pl.ds(g * CHUNK, CHUNK)]], rows[buf],
          sems[buf])

    def drain(g, buf):
      pltpu.make_async_copy(
          table_hbm.at[idx_v.at[pl.ds(g * CHUNK, CHUNK)]], rows[buf],
          sems[buf]).wait()
      pltpu.sync_copy(rows[buf], out_hbm.at[pl.ds(base + g * CHUNK, CHUNK)])

    start(0, 0)
    start(1, 1)

    def body(h, _):
      for b in range(2):
        g = h * 2 + b
        drain(g, b)

        @pl.when(g + 2 < n_chunks)
        def _():
          start(g + 2, b)

      return 0

    lax.fori_loop(0, n_chunks // 2, body, 0)

  return gather_kernel


def kernel(x, table):
  b, l = x.shape
  n = b * l
  flat = x.reshape(n).astype(jnp.int32)
  out = _make_gather(n)(flat, table)
  return out.reshape(b, l, D)


# speedup vs baseline: 1.0179x; 1.0179x over previous
import functools

import jax
import jax.numpy as jnp
from jax import lax
from jax.experimental import pallas as pl
from jax.experimental.pallas import tpu as pltpu
from jax.experimental.pallas import tpu_sc as plsc

NC = 2
NS = 16
NW = NC * NS

D = 64
CHUNK = 128
NBUF = 8
DG = 6


def _make_gather(n_rows: int):
  assert n_rows % (NW * CHUNK * NBUF) == 0
  per_w = n_rows // NW
  n_chunks = per_w // CHUNK

  mesh = plsc.VectorSubcoreMesh(
      core_axis_name="c", subcore_axis_name="s", num_cores=NC,
      num_subcores=NS)

  @functools.partial(
      pl.kernel,
      out_type=jax.ShapeDtypeStruct((n_rows, D), jnp.float32),
      mesh=mesh,
      compiler_params=pltpu.CompilerParams(use_tc_tiling_on_sc=False),
      scratch_types=[
          pltpu.VMEM((per_w,), jnp.int32),
          [pltpu.VMEM((CHUNK, D), jnp.float32) for _ in range(NBUF)],
          [pltpu.SemaphoreType.DMA for _ in range(NBUF)],
          [pltpu.SemaphoreType.DMA for _ in range(NBUF)],
      ],
  )
  def gather_kernel(idx_hbm, table_hbm, out_hbm, idx_v, rows, gsems, ssems):
    wid = lax.axis_index("s") * NC + lax.axis_index("c")
    base = wid * per_w
    pltpu.sync_copy(idx_hbm.at[pl.ds(base, per_w)], idx_v)

    def gather(g, buf):
      return pltpu.make_async_copy(
          table_hbm.at[idx_v.at[pl.ds(g * CHUNK, CHUNK)]], rows[buf],
          gsems[buf])

    def store(g, buf):
      return pltpu.make_async_copy(
          rows[buf], out_hbm.at[pl.ds(base + g * CHUNK, CHUNK)], ssems[buf])

    for h in range(DG):
      gather(h, h % NBUF).start()

    def body(grp, _):
      for j in range(NBUF):
        g = grp * NBUF + j
        h = g + DG
        bh = (j + DG) % NBUF

        @pl.when(h < n_chunks)
        def _():
          @pl.when(h >= NBUF)
          def _():
            store(h - NBUF, bh).wait()

          gather(h, bh).start()

        gather(g, j).wait()
        store(g, j).start()
      return 0

    lax.fori_loop(0, n_chunks // NBUF, body, 0)

    for j in range(NBUF):
      store(n_chunks - NBUF + j, j).wait()

  return gather_kernel


def kernel(x, table):
  b, l = x.shape
  n = b * l
  flat = x.reshape(n).astype(jnp.int32)
  out = _make_gather(n)(flat, table)
  return out.reshape(b, l, D)


# speedup vs baseline: 1.0180x; 1.0001x over previous
import functools

import jax
import jax.numpy as jnp
from jax import lax
from jax.experimental import pallas as pl
from jax.experimental.pallas import tpu as pltpu
from jax.experimental.pallas import tpu_sc as plsc

NC = 2
NS = 16
NW = NC * NS

D = 64
CHUNK = 256
NBUF = 8
DG = 6


def _make_gather(n_rows: int):
  assert n_rows % (NW * CHUNK) == 0
  per_w = n_rows // NW
  n_chunks = per_w // CHUNK
  nbuf = next(d for d in (NBUF, 5, 4, 2, 1) if n_chunks % d == 0)
  dg = max(nbuf - 2, 1)

  mesh = plsc.VectorSubcoreMesh(
      core_axis_name="c", subcore_axis_name="s", num_cores=NC,
      num_subcores=NS)

  @functools.partial(
      pl.kernel,
      out_type=jax.ShapeDtypeStruct((n_rows, D), jnp.float32),
      mesh=mesh,
      compiler_params=pltpu.CompilerParams(use_tc_tiling_on_sc=False),
      scratch_types=[
          pltpu.VMEM((per_w,), jnp.int32),
          [pltpu.VMEM((CHUNK, D), jnp.float32) for _ in range(nbuf)],
          [pltpu.SemaphoreType.DMA for _ in range(nbuf)],
          [pltpu.SemaphoreType.DMA for _ in range(nbuf)],
      ],
  )
  def gather_kernel(idx_hbm, table_hbm, out_hbm, idx_v, rows, gsems, ssems):
    wid = lax.axis_index("s") * NC + lax.axis_index("c")
    base = wid * per_w
    pltpu.sync_copy(idx_hbm.at[pl.ds(base, per_w)], idx_v)

    def gather(g, buf):
      return pltpu.make_async_copy(
          table_hbm.at[idx_v.at[pl.ds(g * CHUNK, CHUNK)]], rows[buf],
          gsems[buf])

    def store(g, buf):
      return pltpu.make_async_copy(
          rows[buf], out_hbm.at[pl.ds(base + g * CHUNK, CHUNK)], ssems[buf])

    for h in range(dg):
      gather(h, h % nbuf).start()

    def body(grp, _):
      for j in range(nbuf):
        g = grp * nbuf + j
        h = g + dg
        bh = (j + dg) % nbuf

        @pl.when(h < n_chunks)
        def _():
          @pl.when(h >= nbuf)
          def _():
            store(h - nbuf, bh).wait()

          gather(h, bh).start()

        gather(g, j).wait()
        store(g, j).start()
      return 0

    lax.fori_loop(0, n_chunks // nbuf, body, 0)

    for j in range(nbuf):
      store(n_chunks - nbuf + j, j % nbuf).wait()

  return gather_kernel


def kernel(x, table):
  b, l = x.shape
  n = b * l
  flat = x.reshape(n).astype(jnp.int32)
  out = _make_gather(n)(flat, table)
  return out.reshape(b, l, D)


# speedup vs baseline: 1.0196x; 1.0016x over previous
import functools

import jax
import jax.numpy as jnp
from jax import lax
from jax.experimental import pallas as pl
from jax.experimental.pallas import tpu as pltpu
from jax.experimental.pallas import tpu_sc as plsc

NC = 2
NS = 16
NW = NC * NS

D = 64
CHUNK = 128
NBUF = 8
DG = 6


def _make_gather(n_rows: int):
  assert n_rows % (NW * CHUNK) == 0
  per_w = n_rows // NW
  n_chunks = per_w // CHUNK
  nbuf = next(d for d in (NBUF, 5, 4, 2, 1) if n_chunks % d == 0)
  dg = max(nbuf - 2, 1)

  mesh = plsc.VectorSubcoreMesh(
      core_axis_name="c", subcore_axis_name="s", num_cores=NC,
      num_subcores=NS)

  @functools.partial(
      pl.kernel,
      out_type=jax.ShapeDtypeStruct((n_rows, D), jnp.float32),
      mesh=mesh,
      compiler_params=pltpu.CompilerParams(use_tc_tiling_on_sc=False),
      scratch_types=[
          pltpu.VMEM((per_w,), jnp.int32),
          [pltpu.VMEM((CHUNK, D), jnp.float32) for _ in range(nbuf)],
          [pltpu.SemaphoreType.DMA for _ in range(nbuf)],
          [pltpu.SemaphoreType.DMA for _ in range(nbuf)],
      ],
  )
  def gather_kernel(idx_hbm, table_hbm, out_hbm, idx_v, rows, gsems, ssems):
    wid = lax.axis_index("s") * NC + lax.axis_index("c")
    base = wid * per_w
    pltpu.sync_copy(idx_hbm.at[pl.ds(base, per_w)], idx_v)

    def gather(g, buf):
      return pltpu.make_async_copy(
          table_hbm.at[idx_v.at[pl.ds(g * CHUNK, CHUNK)]], rows[buf],
          gsems[buf])

    def store(g, buf):
      return pltpu.make_async_copy(
          rows[buf], out_hbm.at[pl.ds(base + g * CHUNK, CHUNK)], ssems[buf])

    for h in range(dg):
      gather(h, h % nbuf).start()

    def body(grp, _):
      for j in range(nbuf):
        g = grp * nbuf + j
        h = g + dg
        bh = (j + dg) % nbuf

        @pl.when(h < n_chunks)
        def _():
          @pl.when(h >= nbuf)
          def _():
            store(h - nbuf, bh).wait()

          gather(h, bh).start()

        gather(g, j).wait()
        store(g, j).start()
      return 0

    lax.fori_loop(0, n_chunks // nbuf, body, 0)

    for j in range(nbuf):
      store(n_chunks - nbuf + j, j % nbuf).wait()

  return gather_kernel


def kernel(x, table):
  b, l = x.shape
  n = b * l
  flat = x.reshape(n).astype(jnp.int32)
  out = _make_gather(n)(flat, table)
  return out.reshape(b, l, D)
